# baseline (speedup 1.0000x reference)
import jax
import jax.numpy as jnp
import numpy as np
from jax.experimental import pallas as pl

_THRESHOLD = 0.95
_ALPHA = 0.1
_NUM_CLASSES = 1000
_BATCH = 16384

_MISS = np.float32(_ALPHA / _NUM_CLASSES)
_HIT = np.float32(np.float32(1.0 - _ALPHA) + _MISS)

_BLOCK_ROWS = 2048


def _body(x_ref, t_ref, out_ref, mask_ref):
    x = x_ref[...]
    m = jnp.max(x, axis=1, keepdims=True)
    e = jnp.exp(x - m)
    s = jnp.sum(e, axis=1, keepdims=True)
    conf = 1.0 / s
    msk = conf > _THRESHOLD
    idx = jax.lax.broadcasted_iota(jnp.int32, x.shape, 1)
    pred = jnp.min(jnp.where(x == m, idx, _NUM_CLASSES), axis=1, keepdims=True)
    r = x.shape[0]
    t = t_ref[0].reshape(r, 1)
    label = jnp.where(msk, pred, t)
    out_ref[...] = jnp.where(idx == label, _HIT, _MISS)
    mask_ref[0] = msk.astype(jnp.float32).reshape(1, r)


def kernel(logits, targets):
    b, c = logits.shape
    r = min(_BLOCK_ROWS, b)
    nb = b // r
    tgt3 = targets.astype(jnp.int32).reshape(nb, 1, r)
    smooth, mask3 = pl.pallas_call(
        _body,
        grid=(nb,),
        in_specs=[
            pl.BlockSpec((r, c), lambda i: (i, 0)),
            pl.BlockSpec((1, 1, r), lambda i: (i, 0, 0)),
        ],
        out_specs=[
            pl.BlockSpec((r, c), lambda i: (i, 0)),
            pl.BlockSpec((1, 1, r), lambda i: (i, 0, 0)),
        ],
        out_shape=[
            jax.ShapeDtypeStruct((b, c), jnp.float32),
            jax.ShapeDtypeStruct((nb, 1, r), jnp.float32),
        ],
    )(logits, tgt3)
    return smooth, mask3.reshape(b)
